# baseline (speedup 1.0000x reference)
import functools

import jax
import jax.numpy as jnp
from jax import lax
from jax.experimental import pallas as pl
from jax.experimental.pallas import tpu as pltpu
from jax.experimental.pallas import tpu_sc as plsc

_NC = 2
_NS = 16
_NW = _NC * _NS
_L = 16


def _tc_sims_body(q_ref, mem_ref, out_ref, accd0, accd1, accn0, accn1, qs_ref):
    b = pl.program_id(0)
    j = pl.program_id(1)
    nj = pl.num_programs(1)

    x = mem_ref[:, 0, :, :]
    qb = q_ref[0]
    pd = jnp.sum(x * qb[None, :, :], axis=1)
    pn = jnp.sum(x * x, axis=1)
    qp = jnp.sum(qb * qb)

    first = j == 0

    @pl.when(first)
    def _():
        qs_ref[b] = qp

    @pl.when(jnp.logical_not(first))
    def _():
        qs_ref[b] = qs_ref[b] + qp

    for bb, accd, accn in ((0, accd0, accn0), (1, accd1, accn1)):
        @pl.when((b == bb) & first)
        def _(accd=accd, accn=accn):
            accd[...] = pd
            accn[...] = pn

        @pl.when((b == bb) & jnp.logical_not(first))
        def _(accd=accd, accn=accn):
            accd[...] = accd[...] + pd
            accn[...] = accn[...] + pn

    @pl.when((b == 1) & (j == nj - 1))
    def _():
        eps = jnp.float32(1e-8)
        d0 = jnp.sum(accd0[...], axis=1, keepdims=True)
        d1 = jnp.sum(accd1[...], axis=1, keepdims=True)
        n0 = jnp.sum(accn0[...], axis=1, keepdims=True)
        n1 = jnp.sum(accn1[...], axis=1, keepdims=True)
        qn0 = jnp.maximum(jnp.sqrt(qs_ref[0]), eps)
        qn1 = jnp.maximum(jnp.sqrt(qs_ref[1]), eps)
        m0 = jnp.maximum(jnp.sqrt(n0), eps)
        m1 = jnp.maximum(jnp.sqrt(n1), eps)
        out_ref[...] = 0.5 * (d0 / (qn0 * m0) + d1 / (qn1 * m1))


def _tc_sims(q3, mem4, chunk):
    B, R, _ = q3.shape
    M = mem4.shape[0]
    nj = R // chunk
    return pl.pallas_call(
        _tc_sims_body,
        grid=(B, nj),
        in_specs=[
            pl.BlockSpec((1, chunk, 128), lambda b, j: (b, j, 0)),
            pl.BlockSpec((M, 1, chunk, 128), lambda b, j: (0, b, j, 0)),
        ],
        out_specs=pl.BlockSpec((M, 1), lambda b, j: (0, 0)),
        out_shape=jax.ShapeDtypeStruct((M, 1), jnp.float32),
        scratch_shapes=[
            pltpu.VMEM((M, 128), jnp.float32),
            pltpu.VMEM((M, 128), jnp.float32),
            pltpu.VMEM((M, 128), jnp.float32),
            pltpu.VMEM((M, 128), jnp.float32),
            pltpu.SMEM((2,), jnp.float32),
        ],
        compiler_params=pltpu.CompilerParams(
            dimension_semantics=("arbitrary", "arbitrary"),
        ),
    )(q3, mem4)


def _make_sc_retrieve(M, row_len, k_cap):
    assert M % _L == 0 and row_len % _NW == 0
    cols = row_len // _NW
    assert cols % _L == 0 and cols % 8 == 0
    nchunk = M // _L

    mesh = plsc.VectorSubcoreMesh(
        core_axis_name="c", subcore_axis_name="s",
        num_cores=_NC, num_subcores=_NS,
    )

    @functools.partial(
        pl.kernel,
        out_type=jax.ShapeDtypeStruct((row_len,), jnp.float32),
        mesh=mesh,
        scratch_types=[
            pltpu.VMEM((M,), jnp.float32),
            pltpu.VMEM((_L,), jnp.int32),
            pltpu.VMEM((k_cap,), jnp.int32),
            pltpu.VMEM((k_cap,), jnp.float32),
            pltpu.VMEM((cols,), jnp.float32),
            pltpu.VMEM((cols,), jnp.float32),
        ],
        compiler_params=pltpu.CompilerParams(needs_layout_passes=False),
    )
    def sc_retrieve(sims_hbm, kk_hbm, bank_hbm, out_hbm,
                    sims_v, kk_v, idxl_v, wl_v, buf_v, acc_v):
        wid = lax.axis_index("s") * _NC + lax.axis_index("c")
        base = wid * cols

        pltpu.sync_copy(sims_hbm, sims_v)
        pltpu.sync_copy(kk_hbm, kk_v)

        iota = lax.iota(jnp.int32, _L)
        chunks = [sims_v[pl.ds(16 * a, _L)] for a in range(nchunk)]
        kvec = kk_v[...]

        def rank_step(p, ranks):
            splat = plsc.load_gather(sims_v, [jnp.full((_L,), p, jnp.int32)])
            pv = jnp.full((_L,), p, jnp.int32)
            out = []
            for a in range(nchunk):
                m_ids = iota + (16 * a)
                gt = (splat > chunks[a]).astype(jnp.int32)
                eq = ((splat == chunks[a]) & (pv < m_ids)).astype(jnp.int32)
                out.append(ranks[a] + gt + eq)
            return tuple(out)

        zeros_i = jnp.zeros((_L,), jnp.int32)
        ranks = lax.fori_loop(0, M, rank_step, tuple(zeros_i for _ in range(nchunk)))

        gm = chunks[0]
        for a in range(1, nchunk):
            gm = jnp.maximum(gm, chunks[a])
        gmax = jnp.full((_L,), jnp.max(gm))
        sels = [ranks[a] < kvec for a in range(nchunk)]
        exps = [jnp.where(sels[a], jnp.exp(chunks[a] - gmax), jnp.float32(0.0))
                for a in range(nchunk)]
        tot_v = exps[0]
        for a in range(1, nchunk):
            tot_v = tot_v + exps[a]
        tot = jnp.full((_L,), jnp.sum(tot_v))

        for a in range(nchunk):
            pos = jnp.where(sels[a], ranks[a], jnp.int32(k_cap - 1))
            plsc.store_scatter(idxl_v, [pos], iota + (16 * a), mask=sels[a])
            plsc.store_scatter(wl_v, [pos], exps[a] / tot, mask=sels[a])

        def zero_step(c, _):
            acc_v[pl.ds(c * _L, _L)] = jnp.zeros((_L,), jnp.float32)
            return 0

        lax.fori_loop(0, cols // _L, zero_step, 0)

        k_s = jnp.max(kvec)
        ch0 = idxl_v[pl.ds(0, _L)]
        ch1 = idxl_v[pl.ds(_L, _L)]

        def gather_step(i, _):
            iv = jnp.full((_L,), i, jnp.int32)
            contrib = (jnp.where(iota == iv, ch0, 0)
                       + jnp.where(iota + _L == iv, ch1, 0))
            idx_s = jnp.sum(contrib)
            w_splat = plsc.load_gather(wl_v, [iv])
            off = idx_s * row_len + base
            pltpu.sync_copy(bank_hbm.at[pl.ds(off, cols)], buf_v)

            def acc_step(c, _):
                sl = pl.ds(c * _L, _L)
                acc_v[sl] = acc_v[sl] + w_splat * buf_v[sl]
                return 0

            lax.fori_loop(0, cols // _L, acc_step, 0)
            return 0

        lax.fori_loop(0, k_s, gather_step, 0)

        pltpu.sync_copy(acc_v, out_hbm.at[pl.ds(base, cols)])

    return sc_retrieve


def kernel(query_feature, memory_bank, k):
    B, C, H, W = query_feature.shape
    M = memory_bank.shape[0]
    d = C * H * W
    row_len = B * d
    R = d // 128

    q3 = query_feature.reshape(B, R, 128)
    mem4 = memory_bank.reshape(M, B, R, 128)
    sims = _tc_sims(q3, mem4, chunk=64)

    bank_flat = memory_bank.reshape(row_len * M)
    kk = jnp.full((_L,), k, jnp.int32)
    k_cap = 32
    out_flat = _make_sc_retrieve(M, row_len, k_cap)(
        sims.reshape(M), kk, bank_flat)
    return out_flat.reshape(B, C, H, W)


# speedup vs baseline: 6.9949x; 6.9949x over previous
import functools

import jax
import jax.numpy as jnp
from jax import lax
from jax.experimental import pallas as pl
from jax.experimental.pallas import tpu as pltpu
from jax.experimental.pallas import tpu_sc as plsc

_NC = 2
_NS = 16
_NW = _NC * _NS
_L = 16


def _tc_sims_body(q_ref, mem_ref, out_ref, accd0, accd1, accn0, accn1, qs_ref):
    b = pl.program_id(0)
    j = pl.program_id(1)
    nj = pl.num_programs(1)

    x = mem_ref[:, 0, :, :]
    qb = q_ref[0]
    pd = jnp.sum(x * qb[None, :, :], axis=1)
    pn = jnp.sum(x * x, axis=1)
    qp = jnp.sum(qb * qb)

    first = j == 0

    @pl.when(first)
    def _():
        qs_ref[b] = qp

    @pl.when(jnp.logical_not(first))
    def _():
        qs_ref[b] = qs_ref[b] + qp

    for bb, accd, accn in ((0, accd0, accn0), (1, accd1, accn1)):
        @pl.when((b == bb) & first)
        def _(accd=accd, accn=accn):
            accd[...] = pd
            accn[...] = pn

        @pl.when((b == bb) & jnp.logical_not(first))
        def _(accd=accd, accn=accn):
            accd[...] = accd[...] + pd
            accn[...] = accn[...] + pn

    @pl.when((b == 1) & (j == nj - 1))
    def _():
        eps = jnp.float32(1e-8)
        d0 = jnp.sum(accd0[...], axis=1, keepdims=True)
        d1 = jnp.sum(accd1[...], axis=1, keepdims=True)
        n0 = jnp.sum(accn0[...], axis=1, keepdims=True)
        n1 = jnp.sum(accn1[...], axis=1, keepdims=True)
        qn0 = jnp.maximum(jnp.sqrt(qs_ref[0]), eps)
        qn1 = jnp.maximum(jnp.sqrt(qs_ref[1]), eps)
        m0 = jnp.maximum(jnp.sqrt(n0), eps)
        m1 = jnp.maximum(jnp.sqrt(n1), eps)
        out_ref[...] = 0.5 * (d0 / (qn0 * m0) + d1 / (qn1 * m1))


def _tc_sims(q3, mem4, chunk):
    B, R, _ = q3.shape
    M = mem4.shape[0]
    nj = R // chunk
    return pl.pallas_call(
        _tc_sims_body,
        grid=(B, nj),
        in_specs=[
            pl.BlockSpec((1, chunk, 128), lambda b, j: (b, j, 0)),
            pl.BlockSpec((M, 1, chunk, 128), lambda b, j: (0, b, j, 0)),
        ],
        out_specs=pl.BlockSpec((M, 1), lambda b, j: (0, 0)),
        out_shape=jax.ShapeDtypeStruct((M, 1), jnp.float32),
        scratch_shapes=[
            pltpu.VMEM((M, 128), jnp.float32),
            pltpu.VMEM((M, 128), jnp.float32),
            pltpu.VMEM((M, 128), jnp.float32),
            pltpu.VMEM((M, 128), jnp.float32),
            pltpu.SMEM((2,), jnp.float32),
        ],
        compiler_params=pltpu.CompilerParams(
            dimension_semantics=("arbitrary", "arbitrary"),
        ),
    )(q3, mem4)


def _make_sc_retrieve(M, row_len, k_cap):
    assert M % _L == 0 and row_len % _NW == 0
    cols = row_len // _NW
    assert cols % _L == 0 and cols % 8 == 0
    nchunk = M // _L

    mesh = plsc.VectorSubcoreMesh(
        core_axis_name="c", subcore_axis_name="s",
        num_cores=_NC, num_subcores=_NS,
    )

    @functools.partial(
        pl.kernel,
        out_type=jax.ShapeDtypeStruct((row_len,), jnp.float32),
        mesh=mesh,
        scratch_types=[
            pltpu.VMEM((M,), jnp.float32),
            pltpu.VMEM((_L,), jnp.int32),
            pltpu.VMEM((k_cap,), jnp.int32),
            pltpu.VMEM((k_cap,), jnp.float32),
            pltpu.VMEM((cols,), jnp.float32),
            pltpu.VMEM((cols,), jnp.float32),
        ],
        compiler_params=pltpu.CompilerParams(needs_layout_passes=False),
    )
    def sc_retrieve(sims_hbm, kk_hbm, bank_hbm, out_hbm,
                    sims_v, kk_v, idxl_v, wl_v, buf_v, acc_v):
        wid = lax.axis_index("s") * _NC + lax.axis_index("c")
        base = wid * cols

        pltpu.sync_copy(sims_hbm, sims_v)
        pltpu.sync_copy(kk_hbm, kk_v)

        iota = lax.iota(jnp.int32, _L)
        chunks = [sims_v[pl.ds(16 * a, _L)] for a in range(nchunk)]
        kvec = kk_v[...]

        def rank_step(p, ranks):
            splat = plsc.load_gather(sims_v, [jnp.full((_L,), p, jnp.int32)])
            pv = jnp.full((_L,), p, jnp.int32)
            out = []
            for a in range(nchunk):
                m_ids = iota + (16 * a)
                gt = (splat > chunks[a]).astype(jnp.int32)
                eq = ((splat == chunks[a]) & (pv < m_ids)).astype(jnp.int32)
                out.append(ranks[a] + gt + eq)
            return tuple(out)

        zeros_i = jnp.zeros((_L,), jnp.int32)
        ranks = lax.fori_loop(0, M, rank_step, tuple(zeros_i for _ in range(nchunk)))

        gm = chunks[0]
        for a in range(1, nchunk):
            gm = jnp.maximum(gm, chunks[a])
        gmax = jnp.full((_L,), jnp.max(gm))
        sels = [ranks[a] < kvec for a in range(nchunk)]
        exps = [jnp.where(sels[a], jnp.exp(chunks[a] - gmax), jnp.float32(0.0))
                for a in range(nchunk)]
        tot_v = exps[0]
        for a in range(1, nchunk):
            tot_v = tot_v + exps[a]
        tot = jnp.full((_L,), jnp.sum(tot_v))

        for a in range(nchunk):
            pos = jnp.where(sels[a], ranks[a], jnp.int32(k_cap - 1))
            plsc.store_scatter(idxl_v, [pos], iota + (16 * a), mask=sels[a])
            plsc.store_scatter(wl_v, [pos], exps[a] / tot, mask=sels[a])

        def zero_step(c, _):
            acc_v[pl.ds(c * _L, _L)] = jnp.zeros((_L,), jnp.float32)
            return 0

        lax.fori_loop(0, cols // _L, zero_step, 0)

        k_s = jnp.max(kvec)
        ch0 = idxl_v[pl.ds(0, _L)]
        ch1 = idxl_v[pl.ds(_L, _L)]

        def gather_step(i, _):
            iv = jnp.full((_L,), i, jnp.int32)
            contrib = (jnp.where(iota == iv, ch0, 0)
                       + jnp.where(iota + _L == iv, ch1, 0))
            idx_s = jnp.sum(contrib)
            w_splat = plsc.load_gather(wl_v, [iv])
            off = idx_s * row_len + base
            pltpu.sync_copy(bank_hbm.at[pl.ds(off, cols)], buf_v)

            def acc_step(c, _):
                sl = pl.ds(c * _L, _L)
                acc_v[sl] = acc_v[sl] + w_splat * buf_v[sl]
                return 0

            lax.fori_loop(0, cols // _L, acc_step, 0)
            return 0

        lax.fori_loop(0, k_s, gather_step, 0)

        pltpu.sync_copy(acc_v, out_hbm.at[pl.ds(base, cols)])

    return sc_retrieve


def _to_lanes(x):
    *lead, h, w, c = x.shape
    n = len(lead)
    y = x.reshape(*lead, h, w // 8, 8, c // 128, 128)
    perm = tuple(range(n)) + (n, n + 1, n + 3, n + 2, n + 4)
    y = y.transpose(*perm)
    return y.reshape(*lead, h * (w // 8) * (c // 128) * 8, 128)


def _from_lanes(x, H, W, C):
    *lead, _, _ = x.shape
    n = len(lead)
    y = x.reshape(*lead, H, W // 8, C // 128, 8, 128)
    perm = tuple(range(n)) + (n, n + 1, n + 3, n + 2, n + 4)
    y = y.transpose(*perm)
    return y.reshape(*lead, H, W, C)


def kernel(query_feature, memory_bank, k):
    B, C, H, W = query_feature.shape
    M = memory_bank.shape[0]
    d = C * H * W
    row_len = B * d
    R = d // 128

    q3 = _to_lanes(query_feature.transpose(0, 2, 3, 1))
    mem4 = _to_lanes(memory_bank.transpose(0, 1, 3, 4, 2))
    sims = _tc_sims(q3, mem4, chunk=64)

    bank_flat = mem4.reshape(row_len * M)
    kk = jnp.full((_L,), k, jnp.int32)
    k_cap = 32
    out_flat = _make_sc_retrieve(M, row_len, k_cap)(
        sims.reshape(M), kk, bank_flat)
    out = _from_lanes(out_flat.reshape(B, R, 128), H, W, C)
    return out.transpose(0, 3, 1, 2)
